# baseline (speedup 1.0000x reference)
import functools

import jax
import jax.numpy as jnp
from jax import lax
from jax.experimental import pallas as pl
from jax.experimental.pallas import tpu as pltpu
from jax.experimental.pallas import tpu_sc as plsc

NC = 2
NS = 16
NW = NC * NS
L = 16

K_EDGE = 128

SELU_SCALE = 1.0507009873554805
SELU_ALPHA = 1.6732632423543772


def _sc_sparse_matvec(xT, meta, n, c_per_w):
  nchunk = c_per_w // K_EDGE
  assert nchunk % 2 == 0 and nchunk >= 4
  mrow = 3 * K_EDGE
  mesh = plsc.VectorSubcoreMesh(core_axis_name="c", subcore_axis_name="s")

  @functools.partial(
      pl.kernel,
      out_type=jax.ShapeDtypeStruct((NW, n * L), jnp.float32),
      mesh=mesh,
      scratch_types=[
          pltpu.VMEM((n * L,), jnp.float32),
          pltpu.VMEM((nchunk * mrow,), jnp.int32),
          pltpu.VMEM((2, K_EDGE, L), jnp.float32),
          pltpu.SemaphoreType.DMA,
          pltpu.SemaphoreType.DMA,
          pltpu.SemaphoreType.DMA,
      ],
      compiler_params=pltpu.CompilerParams(needs_layout_passes=False,
                                           use_tc_tiling_on_sc=False),
  )
  def sc_k(xT_h, meta_h, out_h, y_v, meta_v, gbuf, msem, gs0, gs1):
    wid = lax.axis_index("s") * NC + lax.axis_index("c")
    iota16 = lax.iota(jnp.int32, L)
    zero16 = jnp.zeros((L,), jnp.float32)
    gsems = (gs0, gs1)

    def issue_gather(g, b):
      pltpu.async_copy(xT_h.at[meta_v.at[pl.ds(g * mrow, K_EDGE)]],
                       gbuf.at[b], gsems[b])

    def wait_gather(b):
      pltpu.make_async_copy(xT_h.at[pl.ds(0, K_EDGE)],
                            gbuf.at[b], gsems[b]).wait()

    dnums = lax.GatherDimensionNumbers(
        offset_dims=(), collapsed_slice_dims=(0,), start_index_map=(0,))

    def splat(vec, e):
      idxe = jnp.full((L, 1), e, jnp.int32)
      return lax.gather(vec, idxe, dnums, (1,),
                        mode=lax.GatherScatterMode.PROMISE_IN_BOUNDS)

    def compute(g, b):
      def group_body(q, _):
        mb = g * mrow
        rbase16 = meta_v[pl.ds(mb + K_EDGE + q * L, L)]
        vals16 = plsc.bitcast(meta_v[pl.ds(mb + 2 * K_EDGE + q * L, L)],
                              jnp.float32)
        for e in range(L):
          vs = splat(vals16, e)
          rs = splat(rbase16, e)
          xrow = gbuf[b, q * L + e]
          plsc.addupdate_scatter(y_v, [rs + iota16], xrow * vs)
        return 0
      lax.fori_loop(0, K_EDGE // L, group_body, 0, unroll=4)

    pltpu.async_copy(meta_h.at[wid], meta_v, msem)

    def zero_body(i, _):
      base = i * (L * 16)
      for j in range(16):
        y_v[pl.ds(base + j * L, L)] = zero16
      return 0
    lax.fori_loop(0, (n * L) // (L * 16), zero_body, 0, unroll=4)

    pltpu.make_async_copy(meta_h.at[wid], meta_v, msem).wait()
    issue_gather(0, 0)

    def pair_body(go, _):
      for b in range(2):
        g = go * 2 + b
        issue_gather(g + 1, 1 - b)
        wait_gather(b)
        compute(g, b)
      return 0
    lax.fori_loop(0, (nchunk - 2) // 2, pair_body, 0)

    issue_gather(nchunk - 1, 1)
    wait_gather(0)
    compute(nchunk - 2, 0)
    wait_gather(1)
    compute(nchunk - 1, 1)

    pltpu.sync_copy(y_v, out_h.at[wid])

  return sc_k(xT, meta)


def _tc_reduce(parts2, bc16, fl16, n):
  def body(p_r, b_r, f_r, o_r):
    s = jnp.sum(p_r[...], axis=0)
    o_r[...] = b_r[...] + s * f_r[...]

  return pl.pallas_call(
      body,
      out_shape=jax.ShapeDtypeStruct((n * L,), jnp.float32),
  )(parts2, bc16, fl16)


def _tc_dense(ys, bcc, flg, weight1, w2r, n, i_blk):
  ni = n // i_blk

  def tc_body(ys_r, bcc_r, flg_r, w1_r, w2r_r, out_r):
    k = pl.program_id(1)
    t = lax.dot_general(w1_r[0], ys_r[...], (((1,), (0,)), ((), ())),
                        preferred_element_type=jnp.float32)
    st = SELU_SCALE * jnp.where(t > 0, t, SELU_ALPHA * (jnp.exp(t) - 1.0))
    p = st * w2r_r[0]

    @pl.when(k == 0)
    def _():
      out_r[...] = p

    @pl.when(k > 0)
    def _():
      out_r[...] += p

    @pl.when(k == 2)
    def _():
      out_r[...] = bcc_r[...] + out_r[...] * flg_r[...]

  return pl.pallas_call(
      tc_body,
      grid=(ni, 3),
      in_specs=[
          pl.BlockSpec((n, L), lambda i, k: (0, 0)),
          pl.BlockSpec((i_blk, 1), lambda i, k: (i, 0)),
          pl.BlockSpec((i_blk, 1), lambda i, k: (i, 0)),
          pl.BlockSpec((1, i_blk, n), lambda i, k: (k, i, 0)),
          pl.BlockSpec((1, i_blk, 1), lambda i, k: (k, i, 0)),
      ],
      out_specs=pl.BlockSpec((i_blk, L), lambda i, k: (i, 0)),
      out_shape=jax.ShapeDtypeStruct((n, L), jnp.float32),
      compiler_params=pltpu.CompilerParams(
          dimension_semantics=("arbitrary", "arbitrary")),
  )(ys, bcc, flg, weight1, w2r)


def kernel(x, bc_value, interior_flag, B_rows, B_cols, B_vals, weight1,
           weight2):
  n = bc_value.shape[0]
  nnz = B_rows.shape[0]

  c_per_w = -(-nnz // (NW * 2 * K_EDGE)) * 2 * K_EDGE
  pad = NW * c_per_w - nnz
  colsP = jnp.concatenate([B_cols, jnp.zeros((pad,), jnp.int32)])
  rowsP = jnp.concatenate([B_rows, jnp.zeros((pad,), jnp.int32)])
  valsP = jnp.concatenate([B_vals, jnp.zeros((pad,), jnp.float32)])

  nch = c_per_w // K_EDGE
  meta = jnp.stack([
      colsP.reshape(NW, nch, K_EDGE),
      (rowsP * L).reshape(NW, nch, K_EDGE),
      lax.bitcast_convert_type(valsP, jnp.int32).reshape(NW, nch, K_EDGE),
  ], axis=2).reshape(NW, nch * 3 * K_EDGE)

  xT = x.T

  partials = _sc_sparse_matvec(xT, meta, n, c_per_w)

  bc16 = jnp.broadcast_to(bc_value[:, None], (n, L)).reshape(n * L)
  fl16 = jnp.broadcast_to(interior_flag[:, None], (n, L)).reshape(n * L)
  yflat = _tc_reduce(partials, bc16, fl16, n)
  ysum = yflat.reshape(n, L)

  bcc = bc_value.reshape(n, 1)
  flg = interior_flag.reshape(n, 1)
  w2r = weight2.T.reshape(3, n, 1)

  outT = _tc_dense(ysum, bcc, flg, weight1, w2r, n, 256)
  return outT.T


# speedup vs baseline: 1.0735x; 1.0735x over previous
import functools

import jax
import jax.numpy as jnp
from jax import lax
from jax.experimental import pallas as pl
from jax.experimental.pallas import tpu as pltpu
from jax.experimental.pallas import tpu_sc as plsc

NC = 2
NS = 16
NW = NC * NS
L = 16

K_EDGE = 128

SELU_SCALE = 1.0507009873554805
SELU_ALPHA = 1.6732632423543772


def _sc_sparse_matvec(xT, meta, n, c_per_w):
  nchunk = c_per_w // K_EDGE
  assert nchunk % 2 == 0 and nchunk >= 4
  mrow = 3 * K_EDGE
  mesh = plsc.VectorSubcoreMesh(core_axis_name="c", subcore_axis_name="s")

  @functools.partial(
      pl.kernel,
      out_type=jax.ShapeDtypeStruct((NW, n * L), jnp.float32),
      mesh=mesh,
      scratch_types=[
          pltpu.VMEM((n * L,), jnp.float32),
          pltpu.VMEM((nchunk * mrow,), jnp.int32),
          pltpu.VMEM((2, K_EDGE, L), jnp.float32),
          pltpu.SemaphoreType.DMA,
          pltpu.SemaphoreType.DMA,
          pltpu.SemaphoreType.DMA,
      ],
      compiler_params=pltpu.CompilerParams(needs_layout_passes=False,
                                           use_tc_tiling_on_sc=False),
  )
  def sc_k(xT_h, meta_h, out_h, y_v, meta_v, gbuf, msem, gs0, gs1):
    wid = lax.axis_index("s") * NC + lax.axis_index("c")
    iota16 = lax.iota(jnp.int32, L)
    zero16 = jnp.zeros((L,), jnp.float32)
    gsems = (gs0, gs1)

    def issue_gather(g, b):
      pltpu.async_copy(xT_h.at[meta_v.at[pl.ds(g * mrow, K_EDGE)]],
                       gbuf.at[b], gsems[b])

    def wait_gather(b):
      pltpu.make_async_copy(xT_h.at[pl.ds(0, K_EDGE)],
                            gbuf.at[b], gsems[b]).wait()

    dnums = lax.GatherDimensionNumbers(
        offset_dims=(), collapsed_slice_dims=(0,), start_index_map=(0,))

    def splat(vec, e):
      idxe = jnp.full((L, 1), e, jnp.int32)
      return lax.gather(vec, idxe, dnums, (1,),
                        mode=lax.GatherScatterMode.PROMISE_IN_BOUNDS)

    def compute(g, b):
      def group_body(q, _):
        mb = g * mrow
        rbase16 = meta_v[pl.ds(mb + K_EDGE + q * L, L)]
        vals16 = plsc.bitcast(meta_v[pl.ds(mb + 2 * K_EDGE + q * L, L)],
                              jnp.float32)
        for e in range(L):
          vs = splat(vals16, e)
          r = rbase16[e]
          xrow = gbuf[b, q * L + e]
          plsc.addupdate(y_v.at[pl.ds(r, L)], xrow * vs)
        return 0
      lax.fori_loop(0, K_EDGE // L, group_body, 0, unroll=4)

    pltpu.async_copy(meta_h.at[wid], meta_v, msem)

    def zero_body(i, _):
      base = i * (L * 16)
      for j in range(16):
        y_v[pl.ds(base + j * L, L)] = zero16
      return 0
    lax.fori_loop(0, (n * L) // (L * 16), zero_body, 0, unroll=4)

    pltpu.make_async_copy(meta_h.at[wid], meta_v, msem).wait()
    issue_gather(0, 0)

    def pair_body(go, _):
      for b in range(2):
        g = go * 2 + b
        issue_gather(g + 1, 1 - b)
        wait_gather(b)
        compute(g, b)
      return 0
    lax.fori_loop(0, (nchunk - 2) // 2, pair_body, 0)

    issue_gather(nchunk - 1, 1)
    wait_gather(0)
    compute(nchunk - 2, 0)
    wait_gather(1)
    compute(nchunk - 1, 1)

    pltpu.sync_copy(y_v, out_h.at[wid])

  return sc_k(xT, meta)


def _tc_reduce(parts2, bc16, fl16, n):
  def body(p_r, b_r, f_r, o_r):
    s = jnp.sum(p_r[...], axis=0)
    o_r[...] = b_r[...] + s * f_r[...]

  return pl.pallas_call(
      body,
      out_shape=jax.ShapeDtypeStruct((n * L,), jnp.float32),
  )(parts2, bc16, fl16)


def _tc_dense(ys, bcc, flg, weight1, w2r, n, i_blk):
  ni = n // i_blk

  def tc_body(ys_r, bcc_r, flg_r, w1_r, w2r_r, out_r):
    a = ys_r[...]
    acc = jnp.zeros((i_blk, L), jnp.float32)
    for k in range(3):
      t = lax.dot_general(w1_r[k], a, (((1,), (0,)), ((), ())),
                          preferred_element_type=jnp.float32)
      st = SELU_SCALE * jnp.where(t > 0, t, SELU_ALPHA * (jnp.exp(t) - 1.0))
      acc = acc + st * w2r_r[k]
    out_r[...] = bcc_r[...] + acc * flg_r[...]

  return pl.pallas_call(
      tc_body,
      grid=(ni,),
      in_specs=[
          pl.BlockSpec((n, L), lambda i: (0, 0)),
          pl.BlockSpec((i_blk, 1), lambda i: (i, 0)),
          pl.BlockSpec((i_blk, 1), lambda i: (i, 0)),
          pl.BlockSpec((3, i_blk, n), lambda i: (0, i, 0)),
          pl.BlockSpec((3, i_blk, 1), lambda i: (0, i, 0)),
      ],
      out_specs=pl.BlockSpec((i_blk, L), lambda i: (i, 0)),
      out_shape=jax.ShapeDtypeStruct((n, L), jnp.float32),
      compiler_params=pltpu.CompilerParams(
          dimension_semantics=("arbitrary",)),
  )(ys, bcc, flg, weight1, w2r)


def kernel(x, bc_value, interior_flag, B_rows, B_cols, B_vals, weight1,
           weight2):
  n = bc_value.shape[0]
  nnz = B_rows.shape[0]

  c_per_w = -(-nnz // (NW * 2 * K_EDGE)) * 2 * K_EDGE
  pad = NW * c_per_w - nnz
  colsP = jnp.concatenate([B_cols, jnp.zeros((pad,), jnp.int32)])
  rowsP = jnp.concatenate([B_rows, jnp.zeros((pad,), jnp.int32)])
  valsP = jnp.concatenate([B_vals, jnp.zeros((pad,), jnp.float32)])

  nch = c_per_w // K_EDGE
  meta = jnp.stack([
      colsP.reshape(NW, nch, K_EDGE),
      (rowsP * L).reshape(NW, nch, K_EDGE),
      lax.bitcast_convert_type(valsP, jnp.int32).reshape(NW, nch, K_EDGE),
  ], axis=2).reshape(NW, nch * 3 * K_EDGE)

  xT = x.T

  partials = _sc_sparse_matvec(xT, meta, n, c_per_w)

  bc16 = jnp.broadcast_to(bc_value[:, None], (n, L)).reshape(n * L)
  fl16 = jnp.broadcast_to(interior_flag[:, None], (n, L)).reshape(n * L)
  yflat = _tc_reduce(partials, bc16, fl16, n)
  ysum = yflat.reshape(n, L)

  bcc = bc_value.reshape(n, 1)
  flg = interior_flag.reshape(n, 1)
  w2r = weight2.T.reshape(3, n, 1)

  outT = _tc_dense(ysum, bcc, flg, weight1, w2r, n, 256)
  return outT.T


# speedup vs baseline: 1.1006x; 1.0252x over previous
import functools

import jax
import jax.numpy as jnp
from jax import lax
from jax.experimental import pallas as pl
from jax.experimental.pallas import tpu as pltpu
from jax.experimental.pallas import tpu_sc as plsc

NC = 2
NS = 16
NW = NC * NS
L = 16

K_EDGE = 128

SELU_SCALE = 1.0507009873554805
SELU_ALPHA = 1.6732632423543772


def _sc_sparse_matvec(xT, meta, n, c_per_w):
  nchunk = c_per_w // K_EDGE
  assert nchunk % 2 == 0 and nchunk >= 4
  mrow = 3 * K_EDGE
  mesh = plsc.VectorSubcoreMesh(core_axis_name="c", subcore_axis_name="s")

  @functools.partial(
      pl.kernel,
      out_type=jax.ShapeDtypeStruct((NW, n * L), jnp.float32),
      mesh=mesh,
      scratch_types=[
          pltpu.VMEM((n * L,), jnp.float32),
          pltpu.VMEM((nchunk * mrow,), jnp.int32),
          pltpu.VMEM((2, K_EDGE, L), jnp.float32),
          pltpu.SemaphoreType.DMA,
          pltpu.SemaphoreType.DMA,
          pltpu.SemaphoreType.DMA,
      ],
      compiler_params=pltpu.CompilerParams(needs_layout_passes=False,
                                           use_tc_tiling_on_sc=False),
  )
  def sc_k(xT_h, meta_h, out_h, y_v, meta_v, gbuf, msem, gs0, gs1):
    wid = lax.axis_index("s") * NC + lax.axis_index("c")
    iota16 = lax.iota(jnp.int32, L)
    zero16 = jnp.zeros((L,), jnp.float32)
    gsems = (gs0, gs1)

    def issue_gather(g, b):
      pltpu.async_copy(xT_h.at[meta_v.at[pl.ds(g * mrow, K_EDGE)]],
                       gbuf.at[b], gsems[b])

    def wait_gather(b):
      pltpu.make_async_copy(xT_h.at[pl.ds(0, K_EDGE)],
                            gbuf.at[b], gsems[b]).wait()

    dnums = lax.GatherDimensionNumbers(
        offset_dims=(), collapsed_slice_dims=(0,), start_index_map=(0,))

    def splat(vec, e):
      idxe = jnp.full((L, 1), e, jnp.int32)
      return lax.gather(vec, idxe, dnums, (1,),
                        mode=lax.GatherScatterMode.PROMISE_IN_BOUNDS)

    def compute(g, b):
      mb = g * mrow

      @plsc.parallel_loop(0, K_EDGE // L, unroll=2)
      def group_body(q):
        rbase16 = meta_v[pl.ds(mb + K_EDGE + q * L, L)]
        vals16 = plsc.bitcast(meta_v[pl.ds(mb + 2 * K_EDGE + q * L, L)],
                              jnp.float32)
        for e in range(L):
          vs = splat(vals16, e)
          rs = splat(rbase16, e)
          xrow = gbuf[b, q * L + e]
          plsc.addupdate_scatter(y_v, [rs + iota16], xrow * vs)

    pltpu.async_copy(meta_h.at[wid], meta_v, msem)

    def zero_body(i, _):
      base = i * (L * 16)
      for j in range(16):
        y_v[pl.ds(base + j * L, L)] = zero16
      return 0
    lax.fori_loop(0, (n * L) // (L * 16), zero_body, 0, unroll=4)

    pltpu.make_async_copy(meta_h.at[wid], meta_v, msem).wait()
    issue_gather(0, 0)

    def pair_body(go, _):
      for b in range(2):
        g = go * 2 + b
        issue_gather(g + 1, 1 - b)
        wait_gather(b)
        compute(g, b)
      return 0
    lax.fori_loop(0, (nchunk - 2) // 2, pair_body, 0)

    issue_gather(nchunk - 1, 1)
    wait_gather(0)
    compute(nchunk - 2, 0)
    wait_gather(1)
    compute(nchunk - 1, 1)

    pltpu.sync_copy(y_v, out_h.at[wid])

  return sc_k(xT, meta)


def _tc_reduce(parts2, bc16, fl16, n):
  def body(p_r, b_r, f_r, o_r):
    s = jnp.sum(p_r[...], axis=0)
    o_r[...] = b_r[...] + s * f_r[...]

  return pl.pallas_call(
      body,
      out_shape=jax.ShapeDtypeStruct((n * L,), jnp.float32),
  )(parts2, bc16, fl16)


def _tc_dense(ys, bcc, flg, weight1, w2r, n, i_blk):
  ni = n // i_blk

  def tc_body(ys_r, bcc_r, flg_r, w1_r, w2r_r, out_r):
    a = ys_r[...]
    acc = jnp.zeros((i_blk, L), jnp.float32)
    for k in range(3):
      t = lax.dot_general(w1_r[k], a, (((1,), (0,)), ((), ())),
                          preferred_element_type=jnp.float32)
      st = SELU_SCALE * jnp.where(t > 0, t, SELU_ALPHA * (jnp.exp(t) - 1.0))
      acc = acc + st * w2r_r[k]
    out_r[...] = bcc_r[...] + acc * flg_r[...]

  return pl.pallas_call(
      tc_body,
      grid=(ni,),
      in_specs=[
          pl.BlockSpec((n, L), lambda i: (0, 0)),
          pl.BlockSpec((i_blk, 1), lambda i: (i, 0)),
          pl.BlockSpec((i_blk, 1), lambda i: (i, 0)),
          pl.BlockSpec((3, i_blk, n), lambda i: (0, i, 0)),
          pl.BlockSpec((3, i_blk, 1), lambda i: (0, i, 0)),
      ],
      out_specs=pl.BlockSpec((i_blk, L), lambda i: (i, 0)),
      out_shape=jax.ShapeDtypeStruct((n, L), jnp.float32),
      compiler_params=pltpu.CompilerParams(
          dimension_semantics=("arbitrary",)),
  )(ys, bcc, flg, weight1, w2r)


def kernel(x, bc_value, interior_flag, B_rows, B_cols, B_vals, weight1,
           weight2):
  n = bc_value.shape[0]
  nnz = B_rows.shape[0]

  c_per_w = -(-nnz // (NW * 2 * K_EDGE)) * 2 * K_EDGE
  pad = NW * c_per_w - nnz
  colsP = jnp.concatenate([B_cols, jnp.zeros((pad,), jnp.int32)])
  rowsP = jnp.concatenate([B_rows, jnp.zeros((pad,), jnp.int32)])
  valsP = jnp.concatenate([B_vals, jnp.zeros((pad,), jnp.float32)])

  nch = c_per_w // K_EDGE
  meta = jnp.stack([
      colsP.reshape(NW, nch, K_EDGE),
      (rowsP * L).reshape(NW, nch, K_EDGE),
      lax.bitcast_convert_type(valsP, jnp.int32).reshape(NW, nch, K_EDGE),
  ], axis=2).reshape(NW, nch * 3 * K_EDGE)

  xT = x.T

  partials = _sc_sparse_matvec(xT, meta, n, c_per_w)

  bc16 = jnp.broadcast_to(bc_value[:, None], (n, L)).reshape(n * L)
  fl16 = jnp.broadcast_to(interior_flag[:, None], (n, L)).reshape(n * L)
  yflat = _tc_reduce(partials, bc16, fl16, n)
  ysum = yflat.reshape(n, L)

  bcc = bc_value.reshape(n, 1)
  flg = interior_flag.reshape(n, 1)
  w2r = weight2.T.reshape(3, n, 1)

  outT = _tc_dense(ysum, bcc, flg, weight1, w2r, n, 256)
  return outT.T


# speedup vs baseline: 1.1108x; 1.0093x over previous
import functools

import jax
import jax.numpy as jnp
from jax import lax
from jax.experimental import pallas as pl
from jax.experimental.pallas import tpu as pltpu
from jax.experimental.pallas import tpu_sc as plsc

NC = 2
NS = 16
NW = NC * NS
L = 16

K_EDGE = 128

SELU_SCALE = 1.0507009873554805
SELU_ALPHA = 1.6732632423543772


def _sc_sparse_matvec(xT, meta, n, c_per_w):
  nchunk = c_per_w // K_EDGE
  assert nchunk % 2 == 0 and nchunk >= 4
  mrow = 3 * K_EDGE
  mesh = plsc.VectorSubcoreMesh(core_axis_name="c", subcore_axis_name="s")

  @functools.partial(
      pl.kernel,
      out_type=jax.ShapeDtypeStruct((NW, n * L), jnp.float32),
      mesh=mesh,
      scratch_types=[
          pltpu.VMEM((n * L,), jnp.float32),
          pltpu.VMEM((nchunk * mrow,), jnp.int32),
          pltpu.VMEM((2, K_EDGE, L), jnp.float32),
          pltpu.SemaphoreType.DMA,
          pltpu.SemaphoreType.DMA,
          pltpu.SemaphoreType.DMA,
      ],
      compiler_params=pltpu.CompilerParams(needs_layout_passes=False,
                                           use_tc_tiling_on_sc=False),
  )
  def sc_k(xT_h, meta_h, out_h, y_v, meta_v, gbuf, msem, gs0, gs1):
    wid = lax.axis_index("s") * NC + lax.axis_index("c")
    iota16 = lax.iota(jnp.int32, L)
    zero16 = jnp.zeros((L,), jnp.float32)
    gsems = (gs0, gs1)

    def issue_gather(g, b):
      pltpu.async_copy(xT_h.at[meta_v.at[pl.ds(g * mrow, K_EDGE)]],
                       gbuf.at[b], gsems[b])

    def wait_gather(b):
      pltpu.make_async_copy(xT_h.at[pl.ds(0, K_EDGE)],
                            gbuf.at[b], gsems[b]).wait()

    dnums = lax.GatherDimensionNumbers(
        offset_dims=(), collapsed_slice_dims=(0,), start_index_map=(0,))

    def splat(vec, e):
      idxe = jnp.full((L, 1), e, jnp.int32)
      return lax.gather(vec, idxe, dnums, (1,),
                        mode=lax.GatherScatterMode.PROMISE_IN_BOUNDS)

    def compute(g, b):
      mb = g * mrow

      @plsc.parallel_loop(0, K_EDGE // L, unroll=4)
      def group_body(q):
        rbase16 = meta_v[pl.ds(mb + K_EDGE + q * L, L)]
        vals16 = plsc.bitcast(meta_v[pl.ds(mb + 2 * K_EDGE + q * L, L)],
                              jnp.float32)
        for e in range(L):
          vs = splat(vals16, e)
          rs = splat(rbase16, e)
          xrow = gbuf[b, q * L + e]
          plsc.addupdate_scatter(y_v, [rs + iota16], xrow * vs)

    pltpu.async_copy(meta_h.at[wid], meta_v, msem)

    @plsc.parallel_loop(0, (n * L) // (L * 16), unroll=2)
    def zero_body(i):
      base = i * (L * 16)
      for j in range(16):
        y_v[pl.ds(base + j * L, L)] = zero16

    pltpu.make_async_copy(meta_h.at[wid], meta_v, msem).wait()
    issue_gather(0, 0)

    def pair_body(go, _):
      for b in range(2):
        g = go * 2 + b
        issue_gather(g + 1, 1 - b)
        wait_gather(b)
        compute(g, b)
      return 0
    lax.fori_loop(0, (nchunk - 2) // 2, pair_body, 0)

    issue_gather(nchunk - 1, 1)
    wait_gather(0)
    compute(nchunk - 2, 0)
    wait_gather(1)
    compute(nchunk - 1, 1)

    pltpu.sync_copy(y_v, out_h.at[wid])

  return sc_k(xT, meta)


def _tc_reduce(parts2, bc16, fl16, n):
  def body(p_r, b_r, f_r, o_r):
    s = jnp.sum(p_r[...], axis=0)
    o_r[...] = b_r[...] + s * f_r[...]

  return pl.pallas_call(
      body,
      out_shape=jax.ShapeDtypeStruct((n * L,), jnp.float32),
  )(parts2, bc16, fl16)


def _tc_dense(ys, bcc, flg, weight1, w2r, n, i_blk):
  ni = n // i_blk

  def tc_body(ys_r, bcc_r, flg_r, w1_r, w2r_r, out_r):
    a = ys_r[...]
    acc = jnp.zeros((i_blk, L), jnp.float32)
    for k in range(3):
      t = lax.dot_general(w1_r[k], a, (((1,), (0,)), ((), ())),
                          preferred_element_type=jnp.float32)
      st = SELU_SCALE * jnp.where(t > 0, t, SELU_ALPHA * (jnp.exp(t) - 1.0))
      acc = acc + st * w2r_r[k]
    out_r[...] = bcc_r[...] + acc * flg_r[...]

  return pl.pallas_call(
      tc_body,
      grid=(ni,),
      in_specs=[
          pl.BlockSpec((n, L), lambda i: (0, 0)),
          pl.BlockSpec((i_blk, 1), lambda i: (i, 0)),
          pl.BlockSpec((i_blk, 1), lambda i: (i, 0)),
          pl.BlockSpec((3, i_blk, n), lambda i: (0, i, 0)),
          pl.BlockSpec((3, i_blk, 1), lambda i: (0, i, 0)),
      ],
      out_specs=pl.BlockSpec((i_blk, L), lambda i: (i, 0)),
      out_shape=jax.ShapeDtypeStruct((n, L), jnp.float32),
      compiler_params=pltpu.CompilerParams(
          dimension_semantics=("arbitrary",)),
  )(ys, bcc, flg, weight1, w2r)


def kernel(x, bc_value, interior_flag, B_rows, B_cols, B_vals, weight1,
           weight2):
  n = bc_value.shape[0]
  nnz = B_rows.shape[0]

  c_per_w = -(-nnz // (NW * 2 * K_EDGE)) * 2 * K_EDGE
  pad = NW * c_per_w - nnz
  colsP = jnp.concatenate([B_cols, jnp.zeros((pad,), jnp.int32)])
  rowsP = jnp.concatenate([B_rows, jnp.zeros((pad,), jnp.int32)])
  valsP = jnp.concatenate([B_vals, jnp.zeros((pad,), jnp.float32)])

  nch = c_per_w // K_EDGE
  meta = jnp.stack([
      colsP.reshape(NW, nch, K_EDGE),
      (rowsP * L).reshape(NW, nch, K_EDGE),
      lax.bitcast_convert_type(valsP, jnp.int32).reshape(NW, nch, K_EDGE),
  ], axis=2).reshape(NW, nch * 3 * K_EDGE)

  xT = x.T

  partials = _sc_sparse_matvec(xT, meta, n, c_per_w)

  bc16 = jnp.broadcast_to(bc_value[:, None], (n, L)).reshape(n * L)
  fl16 = jnp.broadcast_to(interior_flag[:, None], (n, L)).reshape(n * L)
  yflat = _tc_reduce(partials, bc16, fl16, n)
  ysum = yflat.reshape(n, L)

  bcc = bc_value.reshape(n, 1)
  flg = interior_flag.reshape(n, 1)
  w2r = weight2.T.reshape(3, n, 1)

  outT = _tc_dense(ysum, bcc, flg, weight1, w2r, n, 256)
  return outT.T
